# baseline (speedup 1.0000x reference)
import functools

import jax
import jax.numpy as jnp
from jax import lax
from jax.experimental import pallas as pl
from jax.experimental.pallas import tpu as pltpu
from jax.experimental.pallas import tpu_sc as plsc

BATCH = 1024
SEQ = 200
DIM = 300
DIM_P = 304
TOTAL = BATCH * SEQ

CHUNK = 80


@functools.lru_cache(maxsize=None)
def _build(total):
    info = plsc.get_sparse_core_info()
    nw = info.num_cores * info.num_subcores
    b_per_w = total // nw
    n_chunks = b_per_w // CHUNK
    assert n_chunks % 2 == 0
    mesh = plsc.VectorSubcoreMesh(core_axis_name="c", subcore_axis_name="s")

    @functools.partial(
        pl.kernel,
        mesh=mesh,
        compiler_params=pltpu.CompilerParams(use_tc_tiling_on_sc=False),
        out_type=jax.ShapeDtypeStruct((total * DIM,), jnp.float32),
        scratch_types=[
            pltpu.VMEM((b_per_w,), jnp.int32),
            pltpu.VMEM((CHUNK, DIM_P), jnp.float32),
            pltpu.VMEM((CHUNK, DIM_P), jnp.float32),
            pltpu.VMEM((CHUNK * DIM,), jnp.float32),
            pltpu.VMEM((CHUNK * DIM,), jnp.float32),
            pltpu.SemaphoreType.DMA,
            pltpu.SemaphoreType.DMA,
            pltpu.SemaphoreType.DMA,
            pltpu.SemaphoreType.DMA,
        ],
    )
    def gather_kernel(idx_hbm, table_hbm, out_hbm, idx_all, rows0, rows1,
                      flat0, flat1, gsem0, gsem1, osem0, osem1):
        wid = lax.axis_index("s") * info.num_cores + lax.axis_index("c")
        wbase = wid * b_per_w
        rows = (rows0, rows1)
        flats = (flat0, flat1)
        gsems = (gsem0, gsem1)
        osems = (osem0, osem1)

        pltpu.sync_copy(idx_hbm.at[pl.ds(wbase, b_per_w)], idx_all)

        def gather_src(i):
            return table_hbm.at[idx_all.at[pl.ds(i * CHUNK, CHUNK)]]

        def start_gather(i, b):
            pltpu.async_copy(gather_src(i), rows[b], gsems[b])

        def wait_gather(i, b):
            pltpu.make_async_copy(gather_src(i), rows[b], gsems[b]).wait()

        def out_dst(i):
            return out_hbm.at[pl.ds((wbase + i * CHUNK) * DIM, CHUNK * DIM)]

        def repack(b):
            src = rows[b]
            dst = flats[b]

            def row_body(r, carry):
                rb = r * DIM
                for k in range(18):
                    dst[pl.ds(rb + 16 * k, 16)] = src[r, pl.ds(16 * k, 16)]
                dst[pl.ds(rb + 284, 16)] = src[r, pl.ds(284, 16)]
                return carry

            lax.fori_loop(0, CHUNK, row_body, 0)

        start_gather(0, 0)

        def outer(g, carry):
            for b in range(2):
                i = 2 * g + b
                wait_gather(i, b)

                @pl.when(i + 1 < n_chunks)
                def _():
                    start_gather(i + 1, 1 - b)

                @pl.when(i >= 2)
                def _():
                    pltpu.make_async_copy(flats[b], out_dst(i - 2), osems[b]).wait()

                repack(b)
                pltpu.async_copy(flats[b], out_dst(i), osems[b])
            return carry

        lax.fori_loop(0, n_chunks // 2, outer, 0)

        pltpu.make_async_copy(flats[0], out_dst(n_chunks - 2), osems[0]).wait()
        pltpu.make_async_copy(flats[1], out_dst(n_chunks - 1), osems[1]).wait()

    return gather_kernel


def kernel(x, word_vectors):
    idx = x.reshape(-1).astype(jnp.int32)
    table_p = jnp.pad(word_vectors, ((0, 0), (0, DIM_P - DIM)))
    out = _build(TOTAL)(idx, table_p)
    return out.reshape(BATCH, SEQ, DIM)
